# Initial kernel scaffold; baseline (speedup 1.0000x reference)
import jax
import jax.numpy as jnp
from jax.experimental import pallas as pl


def kernel(x, edge_index, W1, b1, W2, b2, W3, b3):
    raise NotImplementedError("write your pallas kernel here")


# speedup vs baseline: 10.3423x; 10.3423x over previous
import functools

import jax
import jax.numpy as jnp
from jax import lax
from jax.experimental import pallas as pl
from jax.experimental.pallas import tpu as pltpu
from jax.experimental.pallas import tpu_sc as plsc

N = 10000
E = 320000
F = 128
L = 128
NC = 2
NS = 16
RT = 2528
ROWS_B = RT // (NC * NS)
ROWS_C = RT // NS
NACC = 10240
ZR = 160
DR = NACC // NS
NP = 10240

_MESH = plsc.VectorSubcoreMesh(
    core_axis_name="c", subcore_axis_name="s", num_cores=NC, num_subcores=NS
)


def _hist_body(dst_hbm, out_hbm, acc, idxv, onesv, sem):
    c = lax.axis_index("c")
    s = lax.axis_index("s")
    wid = c * NS + s
    for j in range(8):
        onesv[pl.ds(j * 16, 16)] = jnp.ones((16,), jnp.float32)
    for j in range(40):
        onesv[pl.ds(128 + j * 16, 16)] = jnp.zeros((16,), jnp.float32)
    pltpu.sync_copy(onesv.at[pl.ds(128, 640)], acc.at[pl.ds(s * 640, 640)])
    plsc.subcore_barrier()

    def step(r, carry):
        pltpu.sync_copy(dst_hbm.at[r], idxv)
        pltpu.sync_copy(onesv.at[pl.ds(0, L)], acc.at[idxv], add=True)
        return carry

    lax.fori_loop(wid * ROWS_B, (wid + 1) * ROWS_B, step, 0)
    plsc.subcore_barrier()
    pltpu.sync_copy(acc.at[pl.ds(s * 640, 640)], out_hbm.at[c, pl.ds(s * 640, 640)])


_sc_hist = pl.kernel(
    _hist_body,
    out_type=jax.ShapeDtypeStruct((NC, NP), jnp.float32),
    mesh=_MESH,
    scratch_types=[
        pltpu.VMEM_SHARED((NP,), jnp.float32),
        pltpu.VMEM((L,), jnp.int32),
        pltpu.VMEM((768,), jnp.float32),
        pltpu.SemaphoreType.DMA,
    ],
)


def _prop_body(per_core, u_hbm, src_hbm, dst_hbm, out_hbm,
               acc, sidx, didx, rows, zbuf, sem):
    c = lax.axis_index("c")
    s = lax.axis_index("s")
    def zrow(r, carry):
        for j in range(8):
            zbuf[r, pl.ds(j * 16, 16)] = jnp.zeros((16,), jnp.float32)
        return carry
    lax.fori_loop(0, ZR, zrow, 0)
    for k in range(NACC // NS // ZR):
        pltpu.sync_copy(zbuf, acc.at[pl.ds(s * (NACC // NS) + k * ZR, ZR)])
    plsc.subcore_barrier()

    if per_core:
        lo = s * ROWS_C
        hi = lo + ROWS_C
    else:
        wid = c * NS + s
        lo = wid * ROWS_B
        hi = lo + ROWS_B

    def step(r, carry):
        if per_core:
            pltpu.sync_copy(src_hbm.at[c, r], sidx)
        else:
            pltpu.sync_copy(src_hbm.at[r], sidx)
        pltpu.sync_copy(dst_hbm.at[r], didx)
        pltpu.async_copy(u_hbm.at[sidx], rows, sem).wait()
        pltpu.sync_copy(rows, acc.at[didx], add=True)
        return carry

    lax.fori_loop(lo, hi, step, 0)
    plsc.subcore_barrier()
    pltpu.sync_copy(acc.at[pl.ds(s * DR, DR)], out_hbm.at[c, pl.ds(s * DR, DR)])


def _make_prop(per_core, n_src_rows):
    return pl.kernel(
        functools.partial(_prop_body, per_core),
        out_type=jax.ShapeDtypeStruct((NC, NACC, F), jnp.float32),
        mesh=_MESH,
        scratch_types=[
            pltpu.VMEM_SHARED((NACC, F), jnp.float32),
            pltpu.VMEM((L,), jnp.int32),
            pltpu.VMEM((L,), jnp.int32),
            pltpu.VMEM((L, F), jnp.float32),
            pltpu.VMEM((ZR, F), jnp.float32),
            pltpu.SemaphoreType.DMA,
        ],
    )


_sc_prop_b = _make_prop(False, N)
_sc_prop_c = _make_prop(True, 2 * N)


_BLK = 1000
_GRID = N // _BLK


def _tc1_body(x_ref, hist_ref, w1_ref, b1_ref,
              y1_ref, u1_ref, invs_ref, invd_ref):
    cnt = hist_ref[0] + hist_ref[1]
    deg = cnt + 1.0
    invs = lax.rsqrt(deg)
    invd = 1.0 / deg
    y1 = jnp.dot(x_ref[...], w1_ref[...],
                 preferred_element_type=jnp.float32) + b1_ref[...]
    y1_ref[...] = y1
    u1_ref[...] = invs * y1
    invs_ref[...] = invs
    invd_ref[...] = invd


def _tc2_body(mp_ref, y1_ref, invs_ref, invd_ref, w2_ref, b2_ref,
              y2_ref, u2_ref):
    invs = invs_ref[...]
    invd = invd_ref[...]
    y1 = y1_ref[...]
    m = invs * (mp_ref[0] + mp_ref[1])
    h_lp = jnp.maximum(m + invd * y1, 0.0)
    h_hp = jnp.maximum(y1 - m, 0.0)
    w2 = w2_ref[...]
    b2 = b2_ref[...]
    y2_lp = jnp.dot(h_lp, w2, preferred_element_type=jnp.float32) + b2
    y2_hp = jnp.dot(h_hp, w2, preferred_element_type=jnp.float32) + b2
    y2_ref[0] = y2_lp
    y2_ref[1] = y2_hp
    u2_ref[0] = invs * y2_lp
    u2_ref[1] = invs * y2_hp


def _tc3_body(zp_ref, y2_ref, invs_ref, invd_ref, w3_ref, b3_ref,
              zlp_ref, zhp_ref, plp_ref, php_ref):
    invs = invs_ref[...]
    invd = invd_ref[...]
    z_lp = invs * zp_ref[0] + invd * y2_ref[0]
    z_hp = y2_ref[1] - invs * zp_ref[1]
    w3 = w3_ref[...]
    b3 = b3_ref[...]
    zlp_ref[...] = z_lp
    zhp_ref[...] = z_hp
    plp_ref[...] = jnp.dot(z_lp, w3, preferred_element_type=jnp.float32) + b3
    php_ref[...] = jnp.dot(z_hp, w3, preferred_element_type=jnp.float32) + b3


def _rows(i):
    return (i, 0)


def _full(i):
    return (0, 0)


def _rows3(i):
    return (0, i, 0)


_spec_nf = pl.BlockSpec((_BLK, F), _rows)
_spec_n1 = pl.BlockSpec((_BLK, 1), _rows)
_spec_2n1 = pl.BlockSpec((NC, _BLK, 1), _rows3)
_spec_2nf = pl.BlockSpec((NC, _BLK, F), _rows3)
_spec_2pf = pl.BlockSpec((NC, _BLK, F), _rows3)
_spec_w = pl.BlockSpec((F, F), _full)
_spec_b = pl.BlockSpec((1, F), _full)

_nf = jax.ShapeDtypeStruct((N, F), jnp.float32)
_n1 = jax.ShapeDtypeStruct((N, 1), jnp.float32)
_2nf = jax.ShapeDtypeStruct((NC, N, F), jnp.float32)

_tc1 = pl.pallas_call(
    _tc1_body,
    grid=(_GRID,),
    in_specs=[_spec_nf, _spec_2n1, _spec_w, _spec_b],
    out_specs=[_spec_nf, _spec_nf, _spec_n1, _spec_n1],
    out_shape=[_nf, _nf, _n1, _n1],
)

_tc2 = pl.pallas_call(
    _tc2_body,
    grid=(_GRID,),
    in_specs=[_spec_2pf, _spec_nf, _spec_n1, _spec_n1, _spec_w, _spec_b],
    out_specs=[_spec_2nf, _spec_2nf],
    out_shape=[_2nf, _2nf],
)

_tc3 = pl.pallas_call(
    _tc3_body,
    grid=(_GRID,),
    in_specs=[_spec_2pf, _spec_2nf, _spec_n1, _spec_n1, _spec_w, _spec_b],
    out_specs=[_spec_nf, _spec_nf, _spec_nf, _spec_nf],
    out_shape=[_nf, _nf, _nf, _nf],
)


@jax.jit
def kernel(x, edge_index, W1, b1, W2, b2, W3, b3):
    pad = RT * L - E
    src = jnp.concatenate(
        [edge_index[0], jnp.zeros((pad,), jnp.int32)]).reshape(RT, L)
    dst = jnp.concatenate(
        [edge_index[1], jnp.full((pad,), N, jnp.int32)]).reshape(RT, L)
    src_c = jnp.stack([src, src + N])

    histp = _sc_hist(dst)
    hist2 = histp[:, :N].reshape(NC, N, 1)

    b1r = b1.reshape(1, F)
    b2r = b2.reshape(1, F)
    b3r = b3.reshape(1, F)

    y1, u1, invs, invd = _tc1(x, hist2, W1, b1r)
    mp = _sc_prop_b(u1, src, dst)
    y2, u2 = _tc2(mp, y1, invs, invd, W2, b2r)
    zp = _sc_prop_c(u2.reshape(NC * N, F), src_c, dst)
    z_lp, z_hp, p_lp, p_hp = _tc3(zp, y2, invs, invd, W3, b3r)
    return (z_lp, z_hp, p_lp, p_hp, p_lp, p_hp)
